# baseline (speedup 1.0000x reference)
import functools

import jax
import jax.numpy as jnp
import numpy as np
from jax import lax
from jax.experimental import pallas as pl
from jax.experimental.pallas import tpu as pltpu
from jax.experimental.pallas import tpu_sc as plsc

N = 10000
D = 128
E = 320000
EPS = 1e-5

CHUNK = 128
NC, NS = 2, 16
NW = NC * NS
NUM_CHUNKS = 2560
E_PAD = NUM_CHUNKS * CHUNK
CHUNKS_PER_TILE = NUM_CHUNKS // NW
ROWS_PER_TILE_SP = 632
N_SP = NS * ROWS_PER_TILE_SP
BLK = 2000
REAL_CHUNKS = E // CHUNK

_PAD_IOTA = np.arange(E_PAD - E, dtype=np.int32)
_PAD_EDGES = np.stack([
    (_PAD_IOTA % N).reshape(NUM_CHUNKS - REAL_CHUNKS, CHUNK),
    (N + _PAD_IOTA % (N_SP - N)).reshape(NUM_CHUNKS - REAL_CHUNKS, CHUNK),
])

def _linrelu_body(x_ref, wt_ref, b_ref, o_ref):
    acc = jnp.dot(x_ref[...].astype(jnp.bfloat16),
                  wt_ref[...].astype(jnp.bfloat16),
                  preferred_element_type=jnp.float32)
    o_ref[...] = jnp.maximum(acc + b_ref[...], 0.0)


def _linrelu(x, wt, b2):
    return pl.pallas_call(
        _linrelu_body,
        grid=(N // BLK,),
        in_specs=[
            pl.BlockSpec((BLK, D), lambda i: (i, 0)),
            pl.BlockSpec((D, D), lambda i: (0, 0)),
            pl.BlockSpec((1, D), lambda i: (0, 0)),
        ],
        out_specs=pl.BlockSpec((BLK, D), lambda i: (i, 0)),
        out_shape=jax.ShapeDtypeStruct((N, D), jnp.float32),
    )(x, wt, b2)


def _norm_body(x_ref, parts_ref, g_ref, bt_ref, o_ref):
    h = x_ref[...] + parts_ref[0] + parts_ref[1]
    ms = jnp.mean(h * h, axis=-1, keepdims=True)
    o_ref[...] = h * lax.rsqrt(ms + EPS) * g_ref[...] + bt_ref[...]


def _norm(x, parts, g2, beta2):
    return pl.pallas_call(
        _norm_body,
        grid=(N // BLK,),
        in_specs=[
            pl.BlockSpec((BLK, D), lambda i: (i, 0)),
            pl.BlockSpec((NC, BLK, D), lambda i: (0, i, 0)),
            pl.BlockSpec((1, D), lambda i: (0, 0)),
            pl.BlockSpec((1, D), lambda i: (0, 0)),
        ],
        out_specs=pl.BlockSpec((BLK, D), lambda i: (i, 0)),
        out_shape=jax.ShapeDtypeStruct((N, D), jnp.float32),
    )(x, parts, g2, beta2)


NB = 2
IB = 40
NIB = CHUNKS_PER_TILE // IB
INNER = IB // NB


@functools.partial(
    pl.kernel,
    out_type=jax.ShapeDtypeStruct((NC, N_SP, D), jnp.float32),
    mesh=plsc.VectorSubcoreMesh(core_axis_name="c", subcore_axis_name="s"),
    scratch_types=[
        pltpu.VMEM((IB, CHUNK), jnp.int32),
        pltpu.VMEM((IB, CHUNK), jnp.int32),
        pltpu.VMEM((NB, CHUNK, D), jnp.float32),
        pltpu.VMEM_SHARED((N_SP, D), jnp.float32),
        pltpu.SemaphoreType.DMA((NB,)),
    ],
)
def _sc_agg(m_hbm, esd_hbm, out_hbm, idx_s, idx_d, rows, shared, sem):
    c = lax.axis_index("c")
    s = lax.axis_index("s")
    wid = c * NS + s

    qb = wid * CHUNKS_PER_TILE
    pltpu.sync_copy(esd_hbm.at[0, pl.ds(qb, IB)], idx_s)
    pltpu.sync_copy(esd_hbm.at[1, pl.ds(qb, IB)], idx_d)

    @pl.loop(0, CHUNK)
    def _zrow(r):
        @pl.loop(0, D, step=16)
        def _zcol(col):
            rows[0, r, pl.ds(col, 16)] = jnp.zeros((16,), jnp.float32)

    zbase = s * ROWS_PER_TILE_SP

    @pl.loop(0, (ROWS_PER_TILE_SP // CHUNK) * CHUNK, step=CHUNK)
    def _zspm(r0):
        pltpu.sync_copy(rows.at[0], shared.at[pl.ds(zbase + r0, CHUNK)])

    _rem = ROWS_PER_TILE_SP % CHUNK
    pltpu.sync_copy(
        rows.at[0].at[pl.ds(0, _rem)],
        shared.at[pl.ds(zbase + (ROWS_PER_TILE_SP // CHUNK) * CHUNK, _rem)],
    )

    for b in range(NB):
        pltpu.async_copy(m_hbm.at[idx_s.at[b]], rows.at[b], sem.at[b])

    plsc.subcore_barrier()

    for blk in range(NIB):
        if blk > 0:
            pltpu.sync_copy(esd_hbm.at[0, pl.ds(qb + blk * IB, IB)], idx_s)
            pltpu.sync_copy(esd_hbm.at[1, pl.ds(qb + blk * IB, IB)], idx_d)
            for b in range(NB):
                pltpu.async_copy(m_hbm.at[idx_s.at[b]], rows.at[b], sem.at[b])

        @pl.loop(0, INNER - 1)
        def _edge(i):
            for b in range(NB):
                pltpu.make_async_copy(
                    m_hbm.at[idx_s.at[b]], rows.at[b], sem.at[b]).wait()
                pltpu.sync_copy(rows.at[b], shared.at[idx_d.at[i * NB + b]],
                                add=True)
                pltpu.async_copy(
                    m_hbm.at[idx_s.at[(i + 1) * NB + b]], rows.at[b],
                    sem.at[b])

        for b in range(NB):
            pltpu.make_async_copy(
                m_hbm.at[idx_s.at[b]], rows.at[b], sem.at[b]).wait()
            pltpu.sync_copy(rows.at[b],
                            shared.at[idx_d.at[(INNER - 1) * NB + b]],
                            add=True)

    plsc.subcore_barrier()

    obase = s * ROWS_PER_TILE_SP
    pltpu.sync_copy(
        shared.at[pl.ds(obase, ROWS_PER_TILE_SP)],
        out_hbm.at[c, pl.ds(obase, ROWS_PER_TILE_SP)],
    )


def kernel(x, edge_index, W, b, g, beta):
    wt = W.T
    b2 = b.reshape(1, D)
    g2 = g.reshape(1, D)
    beta2 = beta.reshape(1, D)
    m = _linrelu(x, wt, b2)
    esd = jnp.concatenate(
        [edge_index.reshape(2, REAL_CHUNKS, CHUNK), jnp.asarray(_PAD_EDGES)],
        axis=1)
    parts = _sc_agg(m, esd)
    return _norm(x, parts, g2, beta2)


# speedup vs baseline: 1.0037x; 1.0037x over previous
import functools

import jax
import jax.numpy as jnp
import numpy as np
from jax import lax
from jax.experimental import pallas as pl
from jax.experimental.pallas import tpu as pltpu
from jax.experimental.pallas import tpu_sc as plsc

N = 10000
D = 128
E = 320000
EPS = 1e-5

CHUNK = 128
NC, NS = 2, 16
NW = NC * NS
NUM_CHUNKS = 2560
E_PAD = NUM_CHUNKS * CHUNK
CHUNKS_PER_TILE = NUM_CHUNKS // NW
ROWS_PER_TILE_SP = 632
N_SP = NS * ROWS_PER_TILE_SP
BLK = 2000
REAL_CHUNKS = E // CHUNK

_PAD_IOTA = np.arange(E_PAD - E, dtype=np.int32)
_PAD_EDGES = np.stack([
    (_PAD_IOTA % N).reshape(NUM_CHUNKS - REAL_CHUNKS, CHUNK),
    (N + _PAD_IOTA % (N_SP - N)).reshape(NUM_CHUNKS - REAL_CHUNKS, CHUNK),
])

def _linrelu_body(x_ref, wt_ref, b_ref, o_ref):
    acc = jnp.dot(x_ref[...], wt_ref[...], preferred_element_type=jnp.float32)
    o_ref[...] = jnp.maximum(acc + b_ref[...], 0.0)


def _linrelu(x, wt, b2):
    return pl.pallas_call(
        _linrelu_body,
        grid=(N // BLK,),
        in_specs=[
            pl.BlockSpec((BLK, D), lambda i: (i, 0)),
            pl.BlockSpec((D, D), lambda i: (0, 0)),
            pl.BlockSpec((1, D), lambda i: (0, 0)),
        ],
        out_specs=pl.BlockSpec((BLK, D), lambda i: (i, 0)),
        out_shape=jax.ShapeDtypeStruct((N, D), jnp.float32),
    )(x, wt, b2)


def _norm_body(x_ref, parts_ref, g_ref, bt_ref, o_ref):
    h = x_ref[...] + parts_ref[0] + parts_ref[1]
    ms = jnp.mean(h * h, axis=-1, keepdims=True)
    o_ref[...] = h * lax.rsqrt(ms + EPS) * g_ref[...] + bt_ref[...]


def _norm(x, parts, g2, beta2):
    return pl.pallas_call(
        _norm_body,
        grid=(N // BLK,),
        in_specs=[
            pl.BlockSpec((BLK, D), lambda i: (i, 0)),
            pl.BlockSpec((NC, BLK, D), lambda i: (0, i, 0)),
            pl.BlockSpec((1, D), lambda i: (0, 0)),
            pl.BlockSpec((1, D), lambda i: (0, 0)),
        ],
        out_specs=pl.BlockSpec((BLK, D), lambda i: (i, 0)),
        out_shape=jax.ShapeDtypeStruct((N, D), jnp.float32),
    )(x, parts, g2, beta2)


NB = 2
IB = 40
NIB = CHUNKS_PER_TILE // IB
INNER = IB // NB


@functools.partial(
    pl.kernel,
    out_type=jax.ShapeDtypeStruct((NC, N_SP, D), jnp.float32),
    mesh=plsc.VectorSubcoreMesh(core_axis_name="c", subcore_axis_name="s"),
    scratch_types=[
        pltpu.VMEM((IB, CHUNK), jnp.int32),
        pltpu.VMEM((IB, CHUNK), jnp.int32),
        pltpu.VMEM((NB, CHUNK, D), jnp.float32),
        pltpu.VMEM_SHARED((N_SP, D), jnp.float32),
        pltpu.SemaphoreType.DMA((NB,)),
    ],
)
def _sc_agg(m_hbm, esd_hbm, out_hbm, idx_s, idx_d, rows, shared, sem):
    c = lax.axis_index("c")
    s = lax.axis_index("s")
    wid = c * NS + s

    qb = wid * CHUNKS_PER_TILE
    pltpu.sync_copy(esd_hbm.at[0, pl.ds(qb, IB)], idx_s)
    pltpu.sync_copy(esd_hbm.at[1, pl.ds(qb, IB)], idx_d)

    @pl.loop(0, CHUNK)
    def _zrow(r):
        @pl.loop(0, D, step=16)
        def _zcol(col):
            rows[0, r, pl.ds(col, 16)] = jnp.zeros((16,), jnp.float32)

    zbase = s * ROWS_PER_TILE_SP

    @pl.loop(0, (ROWS_PER_TILE_SP // CHUNK) * CHUNK, step=CHUNK)
    def _zspm(r0):
        pltpu.sync_copy(rows.at[0], shared.at[pl.ds(zbase + r0, CHUNK)])

    _rem = ROWS_PER_TILE_SP % CHUNK
    pltpu.sync_copy(
        rows.at[0].at[pl.ds(0, _rem)],
        shared.at[pl.ds(zbase + (ROWS_PER_TILE_SP // CHUNK) * CHUNK, _rem)],
    )

    for b in range(NB):
        pltpu.async_copy(m_hbm.at[idx_s.at[b]], rows.at[b], sem.at[b])

    plsc.subcore_barrier()

    for blk in range(NIB):
        if blk > 0:
            pltpu.sync_copy(esd_hbm.at[0, pl.ds(qb + blk * IB, IB)], idx_s)
            pltpu.sync_copy(esd_hbm.at[1, pl.ds(qb + blk * IB, IB)], idx_d)
            for b in range(NB):
                pltpu.async_copy(m_hbm.at[idx_s.at[b]], rows.at[b], sem.at[b])

        @pl.loop(0, INNER - 1)
        def _edge(i):
            for b in range(NB):
                pltpu.make_async_copy(
                    m_hbm.at[idx_s.at[b]], rows.at[b], sem.at[b]).wait()
                pltpu.sync_copy(rows.at[b], shared.at[idx_d.at[i * NB + b]],
                                add=True)
                pltpu.async_copy(
                    m_hbm.at[idx_s.at[(i + 1) * NB + b]], rows.at[b],
                    sem.at[b])

        for b in range(NB):
            pltpu.make_async_copy(
                m_hbm.at[idx_s.at[b]], rows.at[b], sem.at[b]).wait()
            pltpu.sync_copy(rows.at[b],
                            shared.at[idx_d.at[(INNER - 1) * NB + b]],
                            add=True)

    plsc.subcore_barrier()

    obase = s * ROWS_PER_TILE_SP
    pltpu.sync_copy(
        shared.at[pl.ds(obase, ROWS_PER_TILE_SP)],
        out_hbm.at[c, pl.ds(obase, ROWS_PER_TILE_SP)],
    )


def kernel(x, edge_index, W, b, g, beta):
    wt = W.T
    b2 = b.reshape(1, D)
    g2 = g.reshape(1, D)
    beta2 = beta.reshape(1, D)
    m = _linrelu(x, wt, b2)
    esd = jnp.concatenate(
        [edge_index.reshape(2, REAL_CHUNKS, CHUNK), jnp.asarray(_PAD_EDGES)],
        axis=1)
    parts = _sc_agg(m, esd)
    return _norm(x, parts, g2, beta2)


# speedup vs baseline: 1.0261x; 1.0223x over previous
import functools

import jax
import jax.numpy as jnp
import numpy as np
from jax import lax
from jax.experimental import pallas as pl
from jax.experimental.pallas import tpu as pltpu
from jax.experimental.pallas import tpu_sc as plsc

N = 10000
D = 128
E = 320000
EPS = 1e-5

CHUNK = 128
NC, NS = 2, 16
NW = NC * NS
NUM_CHUNKS = 2560
E_PAD = NUM_CHUNKS * CHUNK
CHUNKS_PER_TILE = NUM_CHUNKS // NW
ROWS_PER_TILE_SP = 632
N_SP = NS * ROWS_PER_TILE_SP
BLK = 5000
REAL_CHUNKS = E // CHUNK

_PAD_IOTA = np.arange(E_PAD - E, dtype=np.int32)
_PAD_EDGES = np.stack([
    (_PAD_IOTA % N).reshape(NUM_CHUNKS - REAL_CHUNKS, CHUNK),
    (N + _PAD_IOTA % (N_SP - N)).reshape(NUM_CHUNKS - REAL_CHUNKS, CHUNK),
])

def _linrelu_body(x_ref, wt_ref, b_ref, o_ref):
    acc = jnp.dot(x_ref[...], wt_ref[...], preferred_element_type=jnp.float32)
    o_ref[...] = jnp.maximum(acc + b_ref[...], 0.0)


def _linrelu(x, wt, b2):
    return pl.pallas_call(
        _linrelu_body,
        grid=(N // BLK,),
        in_specs=[
            pl.BlockSpec((BLK, D), lambda i: (i, 0)),
            pl.BlockSpec((D, D), lambda i: (0, 0)),
            pl.BlockSpec((1, D), lambda i: (0, 0)),
        ],
        out_specs=pl.BlockSpec((BLK, D), lambda i: (i, 0)),
        out_shape=jax.ShapeDtypeStruct((N, D), jnp.float32),
    )(x, wt, b2)


def _norm_body(x_ref, parts_ref, g_ref, bt_ref, o_ref):
    h = x_ref[...] + parts_ref[0] + parts_ref[1]
    ms = jnp.mean(h * h, axis=-1, keepdims=True)
    o_ref[...] = h * lax.rsqrt(ms + EPS) * g_ref[...] + bt_ref[...]


def _norm(x, parts, g2, beta2):
    return pl.pallas_call(
        _norm_body,
        grid=(N // BLK,),
        in_specs=[
            pl.BlockSpec((BLK, D), lambda i: (i, 0)),
            pl.BlockSpec((NC, BLK, D), lambda i: (0, i, 0)),
            pl.BlockSpec((1, D), lambda i: (0, 0)),
            pl.BlockSpec((1, D), lambda i: (0, 0)),
        ],
        out_specs=pl.BlockSpec((BLK, D), lambda i: (i, 0)),
        out_shape=jax.ShapeDtypeStruct((N, D), jnp.float32),
    )(x, parts, g2, beta2)


NB = 2
IB = 40
NIB = CHUNKS_PER_TILE // IB
INNER = IB // NB


@functools.partial(
    pl.kernel,
    out_type=jax.ShapeDtypeStruct((NC, N_SP, D), jnp.float32),
    mesh=plsc.VectorSubcoreMesh(core_axis_name="c", subcore_axis_name="s"),
    scratch_types=[
        pltpu.VMEM((IB, CHUNK), jnp.int32),
        pltpu.VMEM((IB, CHUNK), jnp.int32),
        pltpu.VMEM((NB, CHUNK, D), jnp.float32),
        pltpu.VMEM_SHARED((N_SP, D), jnp.float32),
        pltpu.SemaphoreType.DMA((NB,)),
    ],
)
def _sc_agg(m_hbm, esd_hbm, out_hbm, idx_s, idx_d, rows, shared, sem):
    c = lax.axis_index("c")
    s = lax.axis_index("s")
    wid = c * NS + s

    qb = wid * CHUNKS_PER_TILE
    pltpu.sync_copy(esd_hbm.at[0, pl.ds(qb, IB)], idx_s)
    pltpu.sync_copy(esd_hbm.at[1, pl.ds(qb, IB)], idx_d)

    @pl.loop(0, CHUNK)
    def _zrow(r):
        @pl.loop(0, D, step=16)
        def _zcol(col):
            rows[0, r, pl.ds(col, 16)] = jnp.zeros((16,), jnp.float32)

    zbase = s * ROWS_PER_TILE_SP

    @pl.loop(0, (ROWS_PER_TILE_SP // CHUNK) * CHUNK, step=CHUNK)
    def _zspm(r0):
        pltpu.sync_copy(rows.at[0], shared.at[pl.ds(zbase + r0, CHUNK)])

    _rem = ROWS_PER_TILE_SP % CHUNK
    pltpu.sync_copy(
        rows.at[0].at[pl.ds(0, _rem)],
        shared.at[pl.ds(zbase + (ROWS_PER_TILE_SP // CHUNK) * CHUNK, _rem)],
    )

    for b in range(NB):
        pltpu.async_copy(m_hbm.at[idx_s.at[b]], rows.at[b], sem.at[b])

    plsc.subcore_barrier()

    for blk in range(NIB):
        if blk > 0:
            pltpu.sync_copy(esd_hbm.at[0, pl.ds(qb + blk * IB, IB)], idx_s)
            pltpu.sync_copy(esd_hbm.at[1, pl.ds(qb + blk * IB, IB)], idx_d)
            for b in range(NB):
                pltpu.async_copy(m_hbm.at[idx_s.at[b]], rows.at[b], sem.at[b])

        @pl.loop(0, INNER - 1)
        def _edge(i):
            for b in range(NB):
                pltpu.make_async_copy(
                    m_hbm.at[idx_s.at[b]], rows.at[b], sem.at[b]).wait()
                pltpu.sync_copy(rows.at[b], shared.at[idx_d.at[i * NB + b]],
                                add=True)
                pltpu.async_copy(
                    m_hbm.at[idx_s.at[(i + 1) * NB + b]], rows.at[b],
                    sem.at[b])

        for b in range(NB):
            pltpu.make_async_copy(
                m_hbm.at[idx_s.at[b]], rows.at[b], sem.at[b]).wait()
            pltpu.sync_copy(rows.at[b],
                            shared.at[idx_d.at[(INNER - 1) * NB + b]],
                            add=True)

    plsc.subcore_barrier()

    obase = s * ROWS_PER_TILE_SP
    pltpu.sync_copy(
        shared.at[pl.ds(obase, ROWS_PER_TILE_SP)],
        out_hbm.at[c, pl.ds(obase, ROWS_PER_TILE_SP)],
    )


def kernel(x, edge_index, W, b, g, beta):
    wt = W.T
    b2 = b.reshape(1, D)
    g2 = g.reshape(1, D)
    beta2 = beta.reshape(1, D)
    m = _linrelu(x, wt, b2)
    esd = jnp.concatenate(
        [edge_index.reshape(2, REAL_CHUNKS, CHUNK), jnp.asarray(_PAD_EDGES)],
        axis=1)
    parts = _sc_agg(m, esd)
    return _norm(x, parts, g2, beta2)
